# baseline (speedup 1.0000x reference)
import functools

import jax
import jax.numpy as jnp
from jax import lax
from jax.experimental import pallas as pl
from jax.experimental.pallas import tpu as pltpu
from jax.experimental.pallas import tpu_sc as plsc

S = 2048
H = 1024
F = 2048
ER = 7
NE = 8
TOPK = 2
TILE = 256
LT_R = 6144
LT = LT_R + S
NT = LT // TILE
NW = 32


def _router_body(x_ref, w_ref, b_ref, idx_ref, sc_ref):
    x = x_ref[...]
    w = w_ref[...]
    logits = jnp.dot(x, w, preferred_element_type=jnp.float32) + b_ref[...]
    col = lax.broadcasted_iota(jnp.int32, logits.shape, 1)
    p = jax.nn.sigmoid(logits)
    p = jnp.where(col < ER, p, -1.0)
    m1 = jnp.max(p, axis=1, keepdims=True)
    i1 = jnp.min(jnp.where(p >= m1, col, 128), axis=1, keepdims=True)
    p2 = jnp.where(col == i1, -1.0, p)
    m2 = jnp.max(p2, axis=1, keepdims=True)
    i2 = jnp.min(jnp.where(p2 >= m2, col, 128), axis=1, keepdims=True)
    tot = m1 + m2
    idx_ref[...] = jnp.concatenate([i1, i2], axis=1)
    sc_ref[...] = jnp.concatenate([m1 / tot, m2 / tot], axis=1)


def _router(x2d, router_w, routing_bias):
    wpad = jnp.zeros((H, 128), jnp.float32).at[:, :ER].set(router_w.T)
    bpad = jnp.zeros((1, 128), jnp.float32).at[0, :ER].set(routing_bias)
    return pl.pallas_call(
        _router_body,
        out_shape=(
            jax.ShapeDtypeStruct((S, TOPK), jnp.int32),
            jax.ShapeDtypeStruct((S, TOPK), jnp.float32),
        ),
    )(x2d, wpad, bpad)


def _metadata(idx2, sc2):
    e_flat = idx2.reshape(-1).astype(jnp.int32)
    tok_flat = (jnp.arange(S * TOPK, dtype=jnp.int32) // TOPK)
    onehot = (e_flat[:, None] == jnp.arange(ER, dtype=jnp.int32)[None, :])
    onehot = onehot.astype(jnp.int32)
    cum = jnp.cumsum(onehot, axis=0)
    rank = jnp.sum((cum - onehot) * onehot, axis=1)
    counts = cum[-1]
    aligned = ((counts + TILE - 1) // TILE) * TILE
    starts = jnp.concatenate(
        [jnp.zeros((1,), jnp.int32), jnp.cumsum(aligned)[:-1]])
    dest = starts[e_flat] + rank
    tile_starts = starts // TILE
    etile_r = jnp.searchsorted(
        tile_starts, jnp.arange(LT_R // TILE, dtype=jnp.int32), side="right"
    ).astype(jnp.int32) - 1
    etile = jnp.concatenate(
        [etile_r, jnp.full((S // TILE,), ER, jnp.int32)])
    pos = dest.reshape(S, TOPK)
    return tok_flat, dest, etile, pos[:, 0], pos[:, 1]


_GCH = 32
_GNC = (S * TOPK) // (NW * _GCH)


def _sc_dispatch_body(x_hbm, tok_hbm, dest_hbm, out_hbm,
                      tok_all, dest_all, rows0, rows1, rows2,
                      g0, g1, g2, w0, w1, w2):
    wid = lax.axis_index("s") * 2 + lax.axis_index("c")
    per_w = _GNC * _GCH
    base = wid * per_w
    pltpu.sync_copy(tok_hbm.at[pl.ds(base, per_w)], tok_all)
    pltpu.sync_copy(dest_hbm.at[wid], dest_all)
    rows = (rows0, rows1, rows2)
    gsem = (g0, g1, g2)
    wsem = (w0, w1, w2)

    def gather(c):
        return pltpu.async_copy(
            x_hbm.at[tok_all.at[pl.ds(c * _GCH, _GCH)]],
            rows[c % 3], gsem[c % 3])

    cps = [gather(0)] + [None] * (_GNC - 1)
    wbs = [None] * _GNC
    for c in range(_GNC):
        if c >= 2:
            wbs[c - 2].wait()
        if c + 1 < _GNC:
            cps[c + 1] = gather(c + 1)
        cps[c].wait()
        wbs[c] = pltpu.async_copy(
            rows[c % 3], out_hbm.at[dest_all.at[c]], wsem[c % 3])
    wbs[_GNC - 2].wait()
    wbs[_GNC - 1].wait()


def _sc_dispatch(x2d, tok_flat, dest):
    dest3 = dest.reshape(NW, _GNC, _GCH)
    mesh = plsc.VectorSubcoreMesh(core_axis_name="c", subcore_axis_name="s")
    return pl.kernel(
        _sc_dispatch_body,
        out_type=jax.ShapeDtypeStruct((LT_R, H), jnp.float32),
        mesh=mesh,
        scratch_types=[
            pltpu.VMEM((_GNC * _GCH,), jnp.int32),
            pltpu.VMEM((_GNC, _GCH), jnp.int32),
            pltpu.VMEM((_GCH, H), jnp.float32),
            pltpu.VMEM((_GCH, H), jnp.float32),
            pltpu.VMEM((_GCH, H), jnp.float32),
            pltpu.SemaphoreType.DMA,
            pltpu.SemaphoreType.DMA,
            pltpu.SemaphoreType.DMA,
            pltpu.SemaphoreType.DMA,
            pltpu.SemaphoreType.DMA,
            pltpu.SemaphoreType.DMA,
        ],
    )(x2d, tok_flat, dest3)


def _ffn_body(et_ref, xs_ref, wg_ref, wu_ref, wd_ref, ys_ref):
    del et_ref
    xb = xs_ref[...].astype(jnp.bfloat16)
    cn = (((1,), (0,)), ((), ()))
    g = lax.dot_general(xb, wg_ref[0], cn, preferred_element_type=jnp.float32)
    u = lax.dot_general(xb, wu_ref[0], cn, preferred_element_type=jnp.float32)
    h = (jax.nn.silu(g) * u).astype(jnp.bfloat16)
    y = lax.dot_general(h, wd_ref[0], cn, preferred_element_type=jnp.float32)
    ys_ref[...] = y


def _ffn(etile, xs, gate_all, up_all, down_all):
    grid_spec = pltpu.PrefetchScalarGridSpec(
        num_scalar_prefetch=1,
        grid=(NT,),
        in_specs=[
            pl.BlockSpec((TILE, H), lambda t, et: (t, 0)),
            pl.BlockSpec((1, H, F), lambda t, et: (et[t], 0, 0)),
            pl.BlockSpec((1, H, F), lambda t, et: (et[t], 0, 0)),
            pl.BlockSpec((1, F, H), lambda t, et: (et[t], 0, 0)),
        ],
        out_specs=pl.BlockSpec((TILE, H), lambda t, et: (t, 0)),
    )
    return pl.pallas_call(
        _ffn_body,
        grid_spec=grid_spec,
        out_shape=jax.ShapeDtypeStruct((LT, H), jnp.float32),
    )(etile, xs, gate_all, up_all, down_all)


_CCH = 8
_CNC = S // (NW * _CCH)


def _sc_combine_body(ys_hbm, p0_hbm, p1_hbm, s0_hbm, s1_hbm, out_hbm,
                     i0_all, i1_all, w0_all, w1_all,
                     a0, a1, b0, b1, c0, c1, o0, o1,
                     sa0, sa1, sb0, sb1, sc0, sc1, so0, so1):
    wid = lax.axis_index("s") * 2 + lax.axis_index("c")
    per_w = _CNC * _CCH
    base = wid * per_w
    pltpu.sync_copy(p0_hbm.at[pl.ds(base, per_w)], i0_all)
    pltpu.sync_copy(p1_hbm.at[pl.ds(base, per_w)], i1_all)
    pltpu.sync_copy(s0_hbm.at[pl.ds(base, per_w)], w0_all)
    pltpu.sync_copy(s1_hbm.at[pl.ds(base, per_w)], w1_all)
    av, bv, cv, ov = (a0, a1), (b0, b1), (c0, c1), (o0, o1)
    sa, sb, sc, so = (sa0, sa1), (sb0, sb1), (sc0, sc1), (so0, so1)

    def fire(k):
        s = k % 2
        sl = pl.ds(k * _CCH, _CCH)
        return (
            pltpu.async_copy(ys_hbm.at[i0_all.at[sl]], av[s], sa[s]),
            pltpu.async_copy(ys_hbm.at[i1_all.at[sl]], bv[s], sb[s]),
            pltpu.async_copy(
                ys_hbm.at[pl.ds(LT_R + base + k * _CCH, _CCH)], cv[s], sc[s]),
        )

    cps = [fire(0)] + [None] * (_CNC - 1)
    wbs = [None] * _CNC
    for k in range(_CNC):
        s = k % 2
        if k >= 2:
            wbs[k - 2].wait()
        if k + 1 < _CNC:
            cps[k + 1] = fire(k + 1)
        for cp in cps[k]:
            cp.wait()
        for r in range(_CCH):
            w0 = w0_all[k * _CCH + r, :]
            w1 = w1_all[k * _CCH + r, :]

            def vec(j, _):
                sl = pl.ds(j * 16, 16)
                ov[s][r, sl] = (av[s][r, sl] * w0 + bv[s][r, sl] * w1
                                + cv[s][r, sl])
                return 0
            lax.fori_loop(0, H // 16, vec, 0, unroll=4)
        wbs[k] = pltpu.async_copy(
            ov[s], out_hbm.at[pl.ds(base + k * _CCH, _CCH)], so[s])
    wbs[_CNC - 2].wait()
    wbs[_CNC - 1].wait()


def _sc_combine(ys, pos0, pos1, s0, s1):
    mesh = plsc.VectorSubcoreMesh(core_axis_name="c", subcore_axis_name="s")
    return pl.kernel(
        _sc_combine_body,
        out_type=jax.ShapeDtypeStruct((S, H), jnp.float32),
        mesh=mesh,
        scratch_types=(
            [pltpu.VMEM((_CNC * _CCH,), jnp.int32)] * 2
            + [pltpu.VMEM((_CNC * _CCH, 16), jnp.float32)] * 2
            + [pltpu.VMEM((_CCH, H), jnp.float32)] * 8
            + [pltpu.SemaphoreType.DMA] * 8
        ),
    )(ys, pos0, pos1, s0, s1)


def kernel(x, shared_gate, shared_up, shared_down,
           routed_gate, routed_up, routed_down, router_w, routing_bias):
    x2d = x.reshape(S, H)
    idx2, sc2 = _router(x2d, router_w, routing_bias)
    tok_flat, dest, etile, pos0, pos1 = _metadata(idx2, sc2)
    xs = jnp.concatenate([_sc_dispatch(x2d, tok_flat, dest), x2d])
    gate_all = jnp.concatenate(
        [routed_gate, shared_gate]).astype(jnp.bfloat16).transpose(0, 2, 1)
    up_all = jnp.concatenate(
        [routed_up, shared_up]).astype(jnp.bfloat16).transpose(0, 2, 1)
    down_all = jnp.concatenate(
        [routed_down, shared_down]).astype(jnp.bfloat16).transpose(0, 2, 1)
    ys = _ffn(etile, xs, gate_all, up_all, down_all)
    s0x = jnp.broadcast_to(sc2[:, 0:1], (S, 16))
    s1x = jnp.broadcast_to(sc2[:, 1:2], (S, 16))
    out = _sc_combine(ys, pos0, pos1, s0x, s1x)
    return out.reshape(x.shape)


# speedup vs baseline: 1.1506x; 1.1506x over previous
import functools

import jax
import jax.numpy as jnp
from jax import lax
from jax.experimental import pallas as pl
from jax.experimental.pallas import tpu as pltpu
from jax.experimental.pallas import tpu_sc as plsc

S = 2048
H = 1024
F = 2048
ER = 7
NE = 8
TOPK = 2
TILE = 256
LT_R = 6144
LT = LT_R + S
NT = LT // TILE
NW = 32


def _router_body(x_ref, w_ref, b_ref, idx_ref, sc_ref):
    x = x_ref[...]
    w = w_ref[...]
    logits = jnp.dot(x, w, preferred_element_type=jnp.float32) + b_ref[...]
    col = lax.broadcasted_iota(jnp.int32, logits.shape, 1)
    p = jax.nn.sigmoid(logits)
    p = jnp.where(col < ER, p, -1.0)
    m1 = jnp.max(p, axis=1, keepdims=True)
    i1 = jnp.min(jnp.where(p >= m1, col, 128), axis=1, keepdims=True)
    p2 = jnp.where(col == i1, -1.0, p)
    m2 = jnp.max(p2, axis=1, keepdims=True)
    i2 = jnp.min(jnp.where(p2 >= m2, col, 128), axis=1, keepdims=True)
    tot = m1 + m2
    idx_ref[...] = jnp.concatenate([i1, i2], axis=1)
    sc_ref[...] = jnp.concatenate([m1 / tot, m2 / tot], axis=1)


def _router(x2d, router_w, routing_bias):
    wpad = jnp.zeros((H, 128), jnp.float32).at[:, :ER].set(router_w.T)
    bpad = jnp.zeros((1, 128), jnp.float32).at[0, :ER].set(routing_bias)
    return pl.pallas_call(
        _router_body,
        out_shape=(
            jax.ShapeDtypeStruct((S, TOPK), jnp.int32),
            jax.ShapeDtypeStruct((S, TOPK), jnp.float32),
        ),
    )(x2d, wpad, bpad)


def _metadata(idx2, sc2):
    e_flat = idx2.reshape(-1).astype(jnp.int32)
    tok_flat = (jnp.arange(S * TOPK, dtype=jnp.int32) // TOPK)
    onehot = (e_flat[:, None] == jnp.arange(ER, dtype=jnp.int32)[None, :])
    onehot = onehot.astype(jnp.int32)
    cum = jnp.cumsum(onehot, axis=0)
    rank = jnp.sum((cum - onehot) * onehot, axis=1)
    counts = cum[-1]
    aligned = ((counts + TILE - 1) // TILE) * TILE
    starts = jnp.concatenate(
        [jnp.zeros((1,), jnp.int32), jnp.cumsum(aligned)[:-1]])
    dest = starts[e_flat] + rank
    tile_starts = starts // TILE
    tids_r = jnp.arange(LT_R // TILE, dtype=jnp.int32)
    etile_r = jnp.searchsorted(
        tile_starts, tids_r, side="right").astype(jnp.int32) - 1
    n_live = jnp.sum(aligned) // TILE
    dead = tids_r >= n_live
    eidx = jnp.concatenate(
        [jnp.where(dead, ER, etile_r), jnp.full((S // TILE,), ER, jnp.int32)])
    valid = jnp.concatenate(
        [(~dead).astype(jnp.int32), jnp.ones((S // TILE,), jnp.int32)])
    etile = jnp.stack([eidx, valid])
    pos = dest.reshape(S, TOPK)
    return tok_flat, dest, etile, pos[:, 0], pos[:, 1]


_GCH = 32
_GNC = (S * TOPK) // (NW * _GCH)


def _sc_dispatch_body(x_hbm, tok_hbm, dest_hbm, out_hbm,
                      tok_all, dest_all, rows0, rows1, rows2,
                      g0, g1, g2, w0, w1, w2):
    wid = lax.axis_index("s") * 2 + lax.axis_index("c")
    per_w = _GNC * _GCH
    base = wid * per_w
    pltpu.sync_copy(tok_hbm.at[pl.ds(base, per_w)], tok_all)
    pltpu.sync_copy(dest_hbm.at[wid], dest_all)
    rows = (rows0, rows1, rows2)
    gsem = (g0, g1, g2)
    wsem = (w0, w1, w2)

    def gather(c):
        return pltpu.async_copy(
            x_hbm.at[tok_all.at[pl.ds(c * _GCH, _GCH)]],
            rows[c % 3], gsem[c % 3])

    cps = [gather(0)] + [None] * (_GNC - 1)
    wbs = [None] * _GNC
    for c in range(_GNC):
        if c >= 2:
            wbs[c - 2].wait()
        if c + 1 < _GNC:
            cps[c + 1] = gather(c + 1)
        cps[c].wait()
        wbs[c] = pltpu.async_copy(
            rows[c % 3], out_hbm.at[dest_all.at[c]], wsem[c % 3])
    wbs[_GNC - 2].wait()
    wbs[_GNC - 1].wait()


def _sc_dispatch(x2d, tok_flat, dest):
    dest3 = dest.reshape(NW, _GNC, _GCH)
    mesh = plsc.VectorSubcoreMesh(core_axis_name="c", subcore_axis_name="s")
    return pl.kernel(
        _sc_dispatch_body,
        out_type=jax.ShapeDtypeStruct((LT_R, H), jnp.float32),
        mesh=mesh,
        scratch_types=[
            pltpu.VMEM((_GNC * _GCH,), jnp.int32),
            pltpu.VMEM((_GNC, _GCH), jnp.int32),
            pltpu.VMEM((_GCH, H), jnp.float32),
            pltpu.VMEM((_GCH, H), jnp.float32),
            pltpu.VMEM((_GCH, H), jnp.float32),
            pltpu.SemaphoreType.DMA,
            pltpu.SemaphoreType.DMA,
            pltpu.SemaphoreType.DMA,
            pltpu.SemaphoreType.DMA,
            pltpu.SemaphoreType.DMA,
            pltpu.SemaphoreType.DMA,
        ],
    )(x2d, tok_flat, dest3)


def _ffn_body(et_ref, xs_ref, wg_ref, wu_ref, wd_ref, ys_ref):
    t = pl.program_id(0)

    @pl.when(et_ref[1, t] == 1)
    def _():
        xb = xs_ref[...].astype(jnp.bfloat16)
        cn = (((1,), (1,)), ((), ()))
        g = lax.dot_general(
            xb, wg_ref[0], cn, preferred_element_type=jnp.float32)
        u = lax.dot_general(
            xb, wu_ref[0], cn, preferred_element_type=jnp.float32)
        h = (jax.nn.silu(g) * u).astype(jnp.bfloat16)
        y = lax.dot_general(
            h, wd_ref[0], cn, preferred_element_type=jnp.float32)
        ys_ref[...] = y


def _ffn(etile, xs, gate_all, up_all, down_all):
    grid_spec = pltpu.PrefetchScalarGridSpec(
        num_scalar_prefetch=1,
        grid=(NT,),
        in_specs=[
            pl.BlockSpec((TILE, H), lambda t, et: (t, 0)),
            pl.BlockSpec((1, F, H), lambda t, et: (et[0, t], 0, 0)),
            pl.BlockSpec((1, F, H), lambda t, et: (et[0, t], 0, 0)),
            pl.BlockSpec((1, H, F), lambda t, et: (et[0, t], 0, 0)),
        ],
        out_specs=pl.BlockSpec((TILE, H), lambda t, et: (t, 0)),
    )
    return pl.pallas_call(
        _ffn_body,
        grid_spec=grid_spec,
        out_shape=jax.ShapeDtypeStruct((LT, H), jnp.float32),
    )(etile, xs, gate_all, up_all, down_all)


_CCH = 8
_CNC = S // (NW * _CCH)


def _sc_combine_body(ys_hbm, p0_hbm, p1_hbm, s0_hbm, s1_hbm, out_hbm,
                     i0_all, i1_all, w0_all, w1_all,
                     a0, a1, b0, b1, c0, c1, o0, o1,
                     sa0, sa1, sb0, sb1, sc0, sc1, so0, so1):
    wid = lax.axis_index("s") * 2 + lax.axis_index("c")
    per_w = _CNC * _CCH
    base = wid * per_w
    pltpu.sync_copy(p0_hbm.at[pl.ds(base, per_w)], i0_all)
    pltpu.sync_copy(p1_hbm.at[pl.ds(base, per_w)], i1_all)
    pltpu.sync_copy(s0_hbm.at[pl.ds(base, per_w)], w0_all)
    pltpu.sync_copy(s1_hbm.at[pl.ds(base, per_w)], w1_all)
    av, bv, cv, ov = (a0, a1), (b0, b1), (c0, c1), (o0, o1)
    sa, sb, sc, so = (sa0, sa1), (sb0, sb1), (sc0, sc1), (so0, so1)

    def fire(k):
        s = k % 2
        sl = pl.ds(k * _CCH, _CCH)
        return (
            pltpu.async_copy(ys_hbm.at[i0_all.at[sl]], av[s], sa[s]),
            pltpu.async_copy(ys_hbm.at[i1_all.at[sl]], bv[s], sb[s]),
            pltpu.async_copy(
                ys_hbm.at[pl.ds(LT_R + base + k * _CCH, _CCH)], cv[s], sc[s]),
        )

    cps = [fire(0)] + [None] * (_CNC - 1)
    wbs = [None] * _CNC
    for k in range(_CNC):
        s = k % 2
        if k >= 2:
            wbs[k - 2].wait()
        if k + 1 < _CNC:
            cps[k + 1] = fire(k + 1)
        for cp in cps[k]:
            cp.wait()
        for r in range(_CCH):
            w0 = w0_all[k * _CCH + r, :]
            w1 = w1_all[k * _CCH + r, :]

            def vec(j, _):
                sl = pl.ds(j * 16, 16)
                ov[s][r, sl] = (av[s][r, sl] * w0 + bv[s][r, sl] * w1
                                + cv[s][r, sl])
                return 0
            lax.fori_loop(0, H // 16, vec, 0, unroll=4)
        wbs[k] = pltpu.async_copy(
            ov[s], out_hbm.at[pl.ds(base + k * _CCH, _CCH)], so[s])
    wbs[_CNC - 2].wait()
    wbs[_CNC - 1].wait()


def _sc_combine(ys, pos0, pos1, s0, s1):
    mesh = plsc.VectorSubcoreMesh(core_axis_name="c", subcore_axis_name="s")
    return pl.kernel(
        _sc_combine_body,
        out_type=jax.ShapeDtypeStruct((S, H), jnp.float32),
        mesh=mesh,
        scratch_types=(
            [pltpu.VMEM((_CNC * _CCH,), jnp.int32)] * 2
            + [pltpu.VMEM((_CNC * _CCH, 16), jnp.float32)] * 2
            + [pltpu.VMEM((_CCH, H), jnp.float32)] * 8
            + [pltpu.SemaphoreType.DMA] * 8
        ),
    )(ys, pos0, pos1, s0, s1)


def kernel(x, shared_gate, shared_up, shared_down,
           routed_gate, routed_up, routed_down, router_w, routing_bias):
    x2d = x.reshape(S, H)
    idx2, sc2 = _router(x2d, router_w, routing_bias)
    tok_flat, dest, etile, pos0, pos1 = _metadata(idx2, sc2)
    xs = jnp.concatenate([_sc_dispatch(x2d, tok_flat, dest), x2d])
    gate_all = jnp.concatenate([routed_gate, shared_gate]).astype(jnp.bfloat16)
    up_all = jnp.concatenate([routed_up, shared_up]).astype(jnp.bfloat16)
    down_all = jnp.concatenate([routed_down, shared_down]).astype(jnp.bfloat16)
    ys = _ffn(etile, xs, gate_all, up_all, down_all)
    s0x = jnp.broadcast_to(sc2[:, 0:1], (S, 16))
    s1x = jnp.broadcast_to(sc2[:, 1:2], (S, 16))
    out = _sc_combine(ys, pos0, pos1, s0x, s1x)
    return out.reshape(x.shape)
